# Initial kernel scaffold; baseline (speedup 1.0000x reference)
import jax
import jax.numpy as jnp
from jax.experimental import pallas as pl


def kernel(v1, y1, v2, y2, idx1, idx2, memory_v1, memory_v2):
    raise NotImplementedError("write your pallas kernel here")


# speedup vs baseline: 3.8439x; 3.8439x over previous
import functools

import jax
import jax.numpy as jnp
from jax import lax
from jax.experimental import pallas as pl
from jax.experimental.pallas import tpu as pltpu
from jax.experimental.pallas import tpu_sc as plsc

_NC = 2
_NS = 16
_NW = _NC * _NS

_B = 128
_KP = 512
_D = 512
_R = _B * _KP
_RPW = _R // _NW
_CH = 128
_NCHUNK = _RPW // _CH

_T = 0.07
_INV_COUNT = 1.0 / (2 * _B)


def _sc_gather(mem1, mem2, idxf):
    mesh = plsc.VectorSubcoreMesh(core_axis_name="c", subcore_axis_name="s")

    @functools.partial(
        pl.kernel,
        mesh=mesh,
        out_type=(
            jax.ShapeDtypeStruct((_R, _D), jnp.float32),
            jax.ShapeDtypeStruct((_R, _D), jnp.float32),
        ),
        scratch_types=[
            pltpu.VMEM((_CH,), jnp.int32),
            pltpu.VMEM((_CH, _D), jnp.float32),
            pltpu.SemaphoreType.DMA,
        ],
    )
    def k(m1, m2, idx_hbm, out1, out2, idx_v, rows_v, sem):
        wid = lax.axis_index("s") * _NC + lax.axis_index("c")
        base = wid * _RPW
        for bank, (table, out) in enumerate(((m1, out1), (m2, out2))):
            def body(i, carry, table=table, out=out, bank=bank):
                off = base + i * _CH
                pltpu.sync_copy(idx_hbm.at[bank, pl.ds(off, _CH)], idx_v)
                pltpu.async_copy(table.at[idx_v], rows_v, sem).wait()
                pltpu.sync_copy(rows_v, out.at[pl.ds(off, _CH)])
                return carry
            lax.fori_loop(0, _NCHUNK, body, 0)

    return k(mem1, mem2, idxf)


def _tc_loss_body(g1_ref, g2_ref, v1_ref, v2_ref, out_ref):
    b = pl.program_id(0)
    w = jnp.concatenate([g1_ref[...], g2_ref[...]], axis=0)
    vcat = jnp.concatenate([v1_ref[pl.ds(b, 1), :], v2_ref[pl.ds(b, 1), :]],
                           axis=0)
    adc = lax.dot_general(vcat, w, (((1,), (1,)), ((), ())),
                          precision=lax.Precision.HIGHEST,
                          preferred_element_type=jnp.float32) / _T
    m = jnp.max(adc, axis=1, keepdims=True)
    lse = m + jnp.log(jnp.sum(jnp.exp(adc - m), axis=1, keepdims=True))
    col = lax.broadcasted_iota(jnp.int32, adc.shape, 1)
    pos_mask = (col == 0) | (col == _KP)
    pos = jnp.sum(jnp.where(pos_mask, adc, 0.0), axis=1, keepdims=True)
    contrib = jnp.sum(pos * 0.5 - lse)
    prev = jnp.where(b == 0, 0.0, out_ref[0, 0])
    acc = prev + contrib
    out_ref[0, 0] = jnp.where(b == _B - 1, -acc * _INV_COUNT, acc)


def _tc_loss(g1, g2, v1, v2):
    out = pl.pallas_call(
        _tc_loss_body,
        grid=(_B,),
        in_specs=[
            pl.BlockSpec((_KP, _D), lambda b: (b, 0)),
            pl.BlockSpec((_KP, _D), lambda b: (b, 0)),
            pl.BlockSpec((_B, _D), lambda b: (0, 0)),
            pl.BlockSpec((_B, _D), lambda b: (0, 0)),
        ],
        out_specs=pl.BlockSpec((1, 1), lambda b: (0, 0),
                               memory_space=pltpu.SMEM),
        out_shape=jax.ShapeDtypeStruct((1, 1), jnp.float32),
    )(g1, g2, v1, v2)
    return out[0, 0]


def kernel(v1, y1, v2, y2, idx1, idx2, memory_v1, memory_v2):
    idxf = jnp.stack([idx1.reshape(-1), idx2.reshape(-1)])
    g1, g2 = _sc_gather(memory_v1, memory_v2, idxf)
    return _tc_loss(g1, g2, v1, v2)
